# Initial kernel scaffold; baseline (speedup 1.0000x reference)
import jax
import jax.numpy as jnp
from jax.experimental import pallas as pl


def kernel(features, keys_weight, values_weight, index_map, packpad_index, entity_type):
    raise NotImplementedError("write your pallas kernel here")


# speedup vs baseline: 5.1900x; 5.1900x over previous
import jax
import jax.numpy as jnp
from jax import lax
from jax.experimental import pallas as pl
from jax.experimental.pallas import tpu as pltpu
from jax.experimental.pallas import tpu_sc as plsc

_B = 16
_S = 128
_D = 64
_N_TOTAL = 8192
_POSITIONS = 441
_N_ENTITY = 4
_EXT = 10.0
_STRIDE_Y = 21.0
_NC = 2
_NS = 16
_L = 16
_IPW = (_B * _S) // (_NC * _NS)


def _sc_body(features, keys_w, values_w, index_map, packpad, etype,
             keys_out, values_out,
             keys_sh, vals_sh,
             pp_v, imfull_v, etfull_v, im_v, voff_v, feat_v,
             tx_v, ty_v, kidx_v, vidx_v, kbuf, vbuf,
             sem_f, gsem_k, gsem_v, wsem_k, wsem_v):
    sid = lax.axis_index("s")
    wid = sid * _NC + lax.axis_index("c")
    b = wid // 2
    i_base = (wid % 2) * _IPW

    @pl.when(sid == 0)
    def _stage():
        pltpu.sync_copy(keys_w, keys_sh)
        pltpu.sync_copy(values_w, vals_sh)

    pltpu.sync_copy(packpad.at[b], pp_v)
    pltpu.sync_copy(index_map, imfull_v)
    pltpu.sync_copy(etype, etfull_v)
    for c in range(_S // _L):
        sl = pl.ds(c * _L, _L)
        ppc = pp_v[sl]
        im_v[sl] = plsc.load_gather(imfull_v, [ppc])
        voff_v[sl] = plsc.load_gather(etfull_v, [ppc]) * _POSITIONS
    pltpu.async_copy(features.at[im_v], feat_v, sem_f).wait()

    col0 = jnp.zeros((_L,), jnp.int32)
    lane = lax.iota(jnp.int32, _L)
    for c in range(_S // _L):
        sl = pl.ds(c * _L, _L)
        jv = lane + (c * _L)
        tx_v[sl] = plsc.load_gather(feat_v, [jv, col0])
        ty_v[sl] = plsc.load_gather(feat_v, [jv, col0 + 1])

    plsc.subcore_barrier()

    def compute_idx(i, slot):
        isp = jnp.full((_L,), 0, jnp.int32) + i
        xi = plsc.load_gather(tx_v, [isp])
        yi = plsc.load_gather(ty_v, [isp])
        for c in range(_S // _L):
            sl = pl.ds(c * _L, _L)
            dx = tx_v[sl] - xi
            dy = ty_v[sl] - yi
            cx = jnp.maximum(jnp.minimum(jnp.float32(_EXT), dx),
                             jnp.float32(-_EXT))
            cy = jnp.maximum(jnp.minimum(jnp.float32(_EXT), dy),
                             jnp.float32(-_EXT))
            s = (cx + jnp.float32(_EXT)) + (cy + jnp.float32(_EXT)) * jnp.float32(_STRIDE_Y)
            yv = s + jnp.float32(0.5)
            t = yv.astype(jnp.int32)
            tie = (t.astype(jnp.float32) == yv) & ((t & 1) == 1)
            idx = t - jnp.where(tie, 1, 0)
            kidx_v[slot, sl] = idx
            vidx_v[slot, sl] = idx + voff_v[sl]

    def fire_gathers(slot):
        pltpu.async_copy(keys_sh.at[kidx_v.at[slot]], kbuf.at[slot], gsem_k)
        pltpu.async_copy(vals_sh.at[vidx_v.at[slot]], vbuf.at[slot], gsem_v)

    def wait_gathers(slot):
        pltpu.make_async_copy(keys_sh.at[kidx_v.at[slot]], kbuf.at[slot],
                              gsem_k).wait()
        pltpu.make_async_copy(vals_sh.at[vidx_v.at[slot]], vbuf.at[slot],
                              gsem_v).wait()

    def fire_writebacks(slot, i):
        pltpu.async_copy(kbuf.at[slot], keys_out.at[b, i], wsem_k)
        pltpu.async_copy(vbuf.at[slot], values_out.at[b, i], wsem_v)

    def wait_writebacks(slot, i):
        pltpu.make_async_copy(kbuf.at[slot], keys_out.at[b, i], wsem_k).wait()
        pltpu.make_async_copy(vbuf.at[slot], values_out.at[b, i], wsem_v).wait()

    compute_idx(i_base, 0)
    fire_gathers(0)

    @pl.loop(0, _IPW)
    def _row(it):
        slot = lax.rem(it, 2)
        nslot = lax.rem(it + 1, 2)
        i = i_base + it

        @pl.when(it + 1 < _IPW)
        def _prep_next():
            compute_idx(i + 1, nslot)

        wait_gathers(slot)
        fire_writebacks(slot, i)

        @pl.when(it + 1 < _IPW)
        def _next():
            @pl.when(it >= 1)
            def _drain():
                wait_writebacks(nslot, i - 1)
            fire_gathers(nslot)

    wait_writebacks((_IPW - 2) % 2, i_base + _IPW - 2)
    wait_writebacks((_IPW - 1) % 2, i_base + _IPW - 1)


@jax.jit
def kernel(features, keys_weight, values_weight, index_map, packpad_index,
           entity_type):
    mesh = plsc.VectorSubcoreMesh(core_axis_name="c", subcore_axis_name="s")
    out_type = (
        jax.ShapeDtypeStruct((_B, _S, _S, _D), jnp.float32),
        jax.ShapeDtypeStruct((_B, _S, _S, _D), jnp.float32),
    )
    scratch = [
        pltpu.VMEM_SHARED((_POSITIONS, _D), jnp.float32),
        pltpu.VMEM_SHARED((_POSITIONS * _N_ENTITY, _D), jnp.float32),
        pltpu.VMEM((_S,), jnp.int32),
        pltpu.VMEM((_N_TOTAL,), jnp.int32),
        pltpu.VMEM((_N_TOTAL,), jnp.int32),
        pltpu.VMEM((_S,), jnp.int32),
        pltpu.VMEM((_S,), jnp.int32),
        pltpu.VMEM((_S, 8), jnp.float32),
        pltpu.VMEM((_S,), jnp.float32),
        pltpu.VMEM((_S,), jnp.float32),
        pltpu.VMEM((2, _S), jnp.int32),
        pltpu.VMEM((2, _S), jnp.int32),
        pltpu.VMEM((2, _S, _D), jnp.float32),
        pltpu.VMEM((2, _S, _D), jnp.float32),
        pltpu.SemaphoreType.DMA,
        pltpu.SemaphoreType.DMA,
        pltpu.SemaphoreType.DMA,
        pltpu.SemaphoreType.DMA,
        pltpu.SemaphoreType.DMA,
    ]
    run = pl.kernel(_sc_body, out_type=out_type, mesh=mesh,
                    scratch_types=scratch,
                    compiler_params=pltpu.CompilerParams(
                        needs_layout_passes=False,
                        use_tc_tiling_on_sc=False))
    return run(
        features.astype(jnp.float32),
        keys_weight.astype(jnp.float32),
        values_weight.astype(jnp.float32),
        index_map.astype(jnp.int32),
        packpad_index.astype(jnp.int32),
        entity_type.astype(jnp.int32).reshape(_N_TOTAL),
    )
